# baseline (speedup 1.0000x reference)
import jax
import jax.numpy as jnp
from jax.experimental import pallas as pl
from jax.experimental.pallas import tpu as pltpu

E = 8
NC = 80
REG_MAX = 16
C = 192
B = 16
NO = NC + 4 * REG_MAX

BLK = 256
NJ0 = 4096 // BLK
NJ1 = 1024 // BLK
NJ2 = 256 // BLK
NJ = NJ0 + NJ1 + NJ2


def _moe_kernel(idx_ref, x0_ref, x1_ref, x2_ref, w0_ref, w1_ref, w2_ref,
                c0_ref, c1_ref, c2_ref, out_ref):
    j = pl.program_id(1)

    def compute(x_ref, w_ref, c_ref):
        out_ref[0] = (
            jnp.dot(w_ref[0], x_ref[0], preferred_element_type=jnp.float32)
            + c_ref[0]
        )

    @pl.when(j < NJ0)
    def _():
        compute(x0_ref, w0_ref, c0_ref)

    @pl.when(jnp.logical_and(j >= NJ0, j < NJ0 + NJ1))
    def _():
        compute(x1_ref, w1_ref, c1_ref)

    @pl.when(j >= NJ0 + NJ1)
    def _():
        compute(x2_ref, w2_ref, c2_ref)


def kernel(x0, x1, x2, module_indices, W2_0, b2_0, W3_0, b3_0,
           W2_1, b2_1, W3_1, b3_1, W2_2, b2_2, W3_2, b3_2):
    xs0 = x0.reshape(B, C, NJ0 * BLK)
    xs1 = x1.reshape(B, C, NJ1 * BLK)
    xs2 = x2.reshape(B, C, NJ2 * BLK)
    Ws = [jnp.concatenate([w2, w3], axis=1)
          for w2, w3 in ((W2_0, W3_0), (W2_1, W3_1), (W2_2, W3_2))]
    bs = [jnp.concatenate([bb2, bb3], axis=1)[:, :, None]
          for bb2, bb3 in ((b2_0, b3_0), (b2_1, b3_1), (b2_2, b3_2))]
    idx = module_indices.astype(jnp.int32)

    grid_spec = pltpu.PrefetchScalarGridSpec(
        num_scalar_prefetch=1,
        grid=(B, NJ),
        in_specs=[
            pl.BlockSpec((1, C, BLK), lambda b, j, i: (b, 0, jnp.minimum(j, NJ0 - 1))),
            pl.BlockSpec((1, C, BLK), lambda b, j, i: (b, 0, jnp.clip(j - NJ0, 0, NJ1 - 1))),
            pl.BlockSpec((1, C, BLK), lambda b, j, i: (b, 0, 0)),
            pl.BlockSpec((1, NO, C), lambda b, j, i: (i[b], 0, 0)),
            pl.BlockSpec((1, NO, C), lambda b, j, i: (i[b], 0, 0)),
            pl.BlockSpec((1, NO, C), lambda b, j, i: (i[b], 0, 0)),
            pl.BlockSpec((1, NO, 1), lambda b, j, i: (i[b], 0, 0)),
            pl.BlockSpec((1, NO, 1), lambda b, j, i: (i[b], 0, 0)),
            pl.BlockSpec((1, NO, 1), lambda b, j, i: (i[b], 0, 0)),
        ],
        out_specs=pl.BlockSpec((1, NO, BLK), lambda b, j, i: (b, 0, j)),
    )

    return pl.pallas_call(
        _moe_kernel,
        grid_spec=grid_spec,
        out_shape=jax.ShapeDtypeStruct((B, NO, NJ * BLK), jnp.float32),
        compiler_params=pltpu.CompilerParams(
            dimension_semantics=("arbitrary", "arbitrary"),
        ),
    )(idx, xs0, xs1, xs2, Ws[0], Ws[1], Ws[2], bs[0], bs[1], bs[2])


# speedup vs baseline: 1.7503x; 1.7503x over previous
import jax
import jax.numpy as jnp
from jax.experimental import pallas as pl
from jax.experimental.pallas import tpu as pltpu

E = 8
NC = 80
REG_MAX = 16
C = 192
B = 16
NO = NC + 4 * REG_MAX

BLK = 1024
NJ0 = 4096 // BLK
NJ = NJ0 + 2


def _moe_kernel(idx_ref, x0_ref, x1_ref, x2_ref, w0_ref, w1_ref, w2_ref,
                c0_ref, c1_ref, c2_ref, out_ref):
    j = pl.program_id(1)

    @pl.when(j < NJ0)
    def _():
        out_ref[0] = (
            jnp.dot(w0_ref[0], x0_ref[0], preferred_element_type=jnp.float32)
            + c0_ref[0]
        )

    @pl.when(j == NJ0)
    def _():
        out_ref[0] = (
            jnp.dot(w1_ref[0], x1_ref[0], preferred_element_type=jnp.float32)
            + c1_ref[0]
        )

    @pl.when(j == NJ0 + 1)
    def _():
        out_ref[0, :, 0:256] = (
            jnp.dot(w2_ref[0], x2_ref[0], preferred_element_type=jnp.float32)
            + c2_ref[0]
        )


def kernel(x0, x1, x2, module_indices, W2_0, b2_0, W3_0, b3_0,
           W2_1, b2_1, W3_1, b3_1, W2_2, b2_2, W3_2, b3_2):
    xs0 = x0.reshape(B, C, 4096)
    xs1 = x1.reshape(B, C, 1024)
    xs2 = x2.reshape(B, C, 256)
    Ws = [jnp.concatenate([w2, w3], axis=1)
          for w2, w3 in ((W2_0, W3_0), (W2_1, W3_1), (W2_2, W3_2))]
    bs = [jnp.concatenate([bb2, bb3], axis=1)[:, :, None]
          for bb2, bb3 in ((b2_0, b3_0), (b2_1, b3_1), (b2_2, b3_2))]
    idx = module_indices.astype(jnp.int32)

    grid_spec = pltpu.PrefetchScalarGridSpec(
        num_scalar_prefetch=1,
        grid=(B, NJ),
        in_specs=[
            pl.BlockSpec((1, C, BLK), lambda b, j, i: (b, 0, jnp.minimum(j, NJ0 - 1))),
            pl.BlockSpec((1, C, 1024), lambda b, j, i: (b, 0, 0)),
            pl.BlockSpec((1, C, 256), lambda b, j, i: (b, 0, 0)),
            pl.BlockSpec((1, NO, C), lambda b, j, i: (i[b], 0, 0)),
            pl.BlockSpec((1, NO, C), lambda b, j, i: (i[b], 0, 0)),
            pl.BlockSpec((1, NO, C), lambda b, j, i: (i[b], 0, 0)),
            pl.BlockSpec((1, NO, 1), lambda b, j, i: (i[b], 0, 0)),
            pl.BlockSpec((1, NO, 1), lambda b, j, i: (i[b], 0, 0)),
            pl.BlockSpec((1, NO, 1), lambda b, j, i: (i[b], 0, 0)),
        ],
        out_specs=pl.BlockSpec((1, NO, BLK), lambda b, j, i: (b, 0, j)),
    )

    return pl.pallas_call(
        _moe_kernel,
        grid_spec=grid_spec,
        out_shape=jax.ShapeDtypeStruct((B, NO, 5376), jnp.float32),
        compiler_params=pltpu.CompilerParams(
            dimension_semantics=("arbitrary", "arbitrary"),
        ),
    )(idx, xs0, xs1, xs2, Ws[0], Ws[1], Ws[2], bs[0], bs[1], bs[2])


# speedup vs baseline: 1.7514x; 1.0006x over previous
import jax
import jax.numpy as jnp
from jax.experimental import pallas as pl
from jax.experimental.pallas import tpu as pltpu

E = 8
NC = 80
REG_MAX = 16
C = 192
B = 16
NO = NC + 4 * REG_MAX

BLK = 1024
NJ0 = 4096 // BLK
NJ = NJ0 + 2


def _moe_kernel(idx_ref, x0_ref, x1_ref, x2_ref, w0_ref, w1_ref, w2_ref,
                c0_ref, c1_ref, c2_ref, out_ref):
    j = pl.program_id(1)

    def dot16(w_ref, x):
        return jnp.dot(w_ref[0].astype(jnp.bfloat16), x.astype(jnp.bfloat16),
                       preferred_element_type=jnp.float32)

    @pl.when(j < NJ0)
    def _():
        out_ref[0] = dot16(w0_ref, x0_ref[0]) + c0_ref[0]

    @pl.when(j == NJ0)
    def _():
        out_ref[0] = dot16(w1_ref, x1_ref[0]) + c1_ref[0]

    @pl.when(j == NJ0 + 1)
    def _():
        out_ref[0, :, 0:256] = dot16(w2_ref, x2_ref[0]) + c2_ref[0]


def kernel(x0, x1, x2, module_indices, W2_0, b2_0, W3_0, b3_0,
           W2_1, b2_1, W3_1, b3_1, W2_2, b2_2, W3_2, b3_2):
    xs0 = x0.reshape(B, C, 4096)
    xs1 = x1.reshape(B, C, 1024)
    xs2 = x2.reshape(B, C, 256)
    Ws = [jnp.concatenate([w2, w3], axis=1)
          for w2, w3 in ((W2_0, W3_0), (W2_1, W3_1), (W2_2, W3_2))]
    bs = [jnp.concatenate([bb2, bb3], axis=1)[:, :, None]
          for bb2, bb3 in ((b2_0, b3_0), (b2_1, b3_1), (b2_2, b3_2))]
    idx = module_indices.astype(jnp.int32)

    grid_spec = pltpu.PrefetchScalarGridSpec(
        num_scalar_prefetch=1,
        grid=(B, NJ),
        in_specs=[
            pl.BlockSpec((1, C, BLK), lambda b, j, i: (b, 0, jnp.minimum(j, NJ0 - 1))),
            pl.BlockSpec((1, C, 1024), lambda b, j, i: (b, 0, 0)),
            pl.BlockSpec((1, C, 256), lambda b, j, i: (b, 0, 0)),
            pl.BlockSpec((1, NO, C), lambda b, j, i: (i[b], 0, 0)),
            pl.BlockSpec((1, NO, C), lambda b, j, i: (i[b], 0, 0)),
            pl.BlockSpec((1, NO, C), lambda b, j, i: (i[b], 0, 0)),
            pl.BlockSpec((1, NO, 1), lambda b, j, i: (i[b], 0, 0)),
            pl.BlockSpec((1, NO, 1), lambda b, j, i: (i[b], 0, 0)),
            pl.BlockSpec((1, NO, 1), lambda b, j, i: (i[b], 0, 0)),
        ],
        out_specs=pl.BlockSpec((1, NO, BLK), lambda b, j, i: (b, 0, j)),
    )

    return pl.pallas_call(
        _moe_kernel,
        grid_spec=grid_spec,
        out_shape=jax.ShapeDtypeStruct((B, NO, 5376), jnp.float32),
        compiler_params=pltpu.CompilerParams(
            dimension_semantics=("arbitrary", "arbitrary"),
        ),
    )(idx, xs0, xs1, xs2, Ws[0], Ws[1], Ws[2], bs[0], bs[1], bs[2])


# speedup vs baseline: 2.4000x; 1.3704x over previous
import jax
import jax.numpy as jnp
from jax.experimental import pallas as pl
from jax.experimental.pallas import tpu as pltpu

E = 8
NC = 80
REG_MAX = 16
C = 192
B = 16
NO = NC + 4 * REG_MAX
HW0, HW1, HW2 = 4096, 1024, 256
HWT = HW0 + HW1 + HW2


def _moe_kernel(idx_ref, x0_ref, x1_ref, x2_ref, w0_ref, w1_ref, w2_ref,
                c0_ref, c1_ref, c2_ref, out_ref):
    def dot16(w_ref, x_ref):
        return jnp.dot(w_ref[0].astype(jnp.bfloat16),
                       x_ref[0].astype(jnp.bfloat16),
                       preferred_element_type=jnp.float32)

    out_ref[0, :, 0:HW0] = dot16(w0_ref, x0_ref) + c0_ref[0]
    out_ref[0, :, HW0:HW0 + HW1] = dot16(w1_ref, x1_ref) + c1_ref[0]
    out_ref[0, :, HW0 + HW1:HWT] = dot16(w2_ref, x2_ref) + c2_ref[0]


def kernel(x0, x1, x2, module_indices, W2_0, b2_0, W3_0, b3_0,
           W2_1, b2_1, W3_1, b3_1, W2_2, b2_2, W3_2, b3_2):
    xs0 = x0.reshape(B, C, HW0)
    xs1 = x1.reshape(B, C, HW1)
    xs2 = x2.reshape(B, C, HW2)
    Ws = [jnp.concatenate([w2, w3], axis=1)
          for w2, w3 in ((W2_0, W3_0), (W2_1, W3_1), (W2_2, W3_2))]
    bs = [jnp.concatenate([bb2, bb3], axis=1)[:, :, None]
          for bb2, bb3 in ((b2_0, b3_0), (b2_1, b3_1), (b2_2, b3_2))]
    idx = module_indices.astype(jnp.int32)

    grid_spec = pltpu.PrefetchScalarGridSpec(
        num_scalar_prefetch=1,
        grid=(B,),
        in_specs=[
            pl.BlockSpec((1, C, HW0), lambda b, i: (b, 0, 0)),
            pl.BlockSpec((1, C, HW1), lambda b, i: (b, 0, 0)),
            pl.BlockSpec((1, C, HW2), lambda b, i: (b, 0, 0)),
            pl.BlockSpec((1, NO, C), lambda b, i: (i[b], 0, 0)),
            pl.BlockSpec((1, NO, C), lambda b, i: (i[b], 0, 0)),
            pl.BlockSpec((1, NO, C), lambda b, i: (i[b], 0, 0)),
            pl.BlockSpec((1, NO, 1), lambda b, i: (i[b], 0, 0)),
            pl.BlockSpec((1, NO, 1), lambda b, i: (i[b], 0, 0)),
            pl.BlockSpec((1, NO, 1), lambda b, i: (i[b], 0, 0)),
        ],
        out_specs=pl.BlockSpec((1, NO, HWT), lambda b, i: (b, 0, 0)),
    )

    return pl.pallas_call(
        _moe_kernel,
        grid_spec=grid_spec,
        out_shape=jax.ShapeDtypeStruct((B, NO, HWT), jnp.float32),
        compiler_params=pltpu.CompilerParams(
            dimension_semantics=("arbitrary",),
        ),
    )(idx, xs0, xs1, xs2, Ws[0], Ws[1], Ws[2], bs[0], bs[1], bs[2])
